# baseline (speedup 1.0000x reference)
import functools

import jax
import jax.numpy as jnp
import numpy as np
from jax.experimental import pallas as pl
from jax.experimental.pallas import tpu as pltpu

_N = 10000
_F = 128
_ALPHA = 0.2

_NP = 10240
_BR = 512
_BC = 2560
_PR = 512
_HB = _NP // _BR // 2


def _project_kernel(x_ref, w_ref, b_ref, aw_ref, out_ref, st_ref):
    i = pl.program_id(0)
    o = jnp.dot(x_ref[...], w_ref[...], preferred_element_type=jnp.float32)
    o = o + b_ref[...]
    row = i * _PR + jax.lax.broadcasted_iota(jnp.int32, (_PR, 1), 0)
    o = jnp.where(row < _N, o, 0.0)
    col = jax.lax.broadcasted_iota(jnp.int32, (_PR, 2 * _F), 1)
    out_ref[...] = jnp.where(col < _F,
                             jnp.pad(o, ((0, 0), (0, _F))),
                             jnp.where(col == _F, 1.0, 0.0)).astype(jnp.bfloat16)
    st_ref[...] = jnp.dot(o, aw_ref[...], preferred_element_type=jnp.float32)


def _ev(adj, s, t):
    e = s + t
    e = jnp.maximum(e, _ALPHA * e)
    return jnp.exp2(e).astype(jnp.bfloat16) * adj.astype(jnp.bfloat16)


def _gat_kernel(adjA_ref, adjB_ref, sA_ref, sB_ref, t_ref, out_ref,
                yA_ref, yB_ref, accA_ref, accB_ref, *, nj):
    j = pl.program_id(1)

    @pl.when(j == 0)
    def _init():
        accA_ref[...] = jnp.zeros_like(accA_ref)
        accB_ref[...] = jnp.zeros_like(accB_ref)

    t = t_ref[:, pl.ds(j * _BC, _BC)]
    rhs = out_ref[pl.ds(j * _BC, _BC), :]
    evA = _ev(adjA_ref[...], sA_ref[...], t)
    evB = _ev(adjB_ref[...], sB_ref[...], t)

    @pl.when(j < nj - 1)
    def _acc_body():
        accA_ref[...] += jnp.dot(evA, rhs, preferred_element_type=jnp.float32)
        accB_ref[...] += jnp.dot(evB, rhs, preferred_element_type=jnp.float32)

    @pl.when(j == nj - 1)
    def _acc_last():
        col = j * _BC + jax.lax.broadcasted_iota(jnp.int32, (_BR, _BC), 1)
        mask = col < _N
        accA = accA_ref[...] + jnp.dot(jnp.where(mask, evA, jnp.bfloat16(0.0)), rhs,
                                       preferred_element_type=jnp.float32)
        accB = accB_ref[...] + jnp.dot(jnp.where(mask, evB, jnp.bfloat16(0.0)), rhs,
                                       preferred_element_type=jnp.float32)
        rsA = accA[:, _F:_F + 1]
        rsB = accB[:, _F:_F + 1]
        yA_ref[...] = accA[:, :_F] / jnp.where(rsA == 0.0, 1.0, rsA)
        yB_ref[...] = accB[:, :_F] / jnp.where(rsB == 0.0, 1.0, rsB)


def kernel(input, adj, W, b, attn_w):
    aw = attn_w.reshape(_F, 2) * np.float32(np.log2(np.e))
    b2 = b.reshape(1, _F)

    out, st = pl.pallas_call(
        _project_kernel,
        grid=(_NP // _PR,),
        in_specs=[
            pl.BlockSpec((_PR, _F), lambda i: (i, 0)),
            pl.BlockSpec((_F, _F), lambda i: (0, 0)),
            pl.BlockSpec((1, _F), lambda i: (0, 0)),
            pl.BlockSpec((_F, 2), lambda i: (0, 0)),
        ],
        out_specs=[
            pl.BlockSpec((_PR, 2 * _F), lambda i: (i, 0)),
            pl.BlockSpec((_PR, 2), lambda i: (i, 0)),
        ],
        out_shape=[
            jax.ShapeDtypeStruct((_NP, 2 * _F), jnp.bfloat16),
            jax.ShapeDtypeStruct((_NP, 2), jnp.float32),
        ],
    )(input, W, b2, aw)

    s = st[:, 0:1]
    t = st[:, 1:2].T

    nhalf = _NP // 2
    ni, nj = _HB, _NP // _BC
    yA, yB = pl.pallas_call(
        functools.partial(_gat_kernel, nj=nj),
        grid=(ni, nj),
        in_specs=[
            pl.BlockSpec((_BR, _BC), lambda i, j: (i, j)),
            pl.BlockSpec((_BR, _BC), lambda i, j: (i + _HB, j)),
            pl.BlockSpec((_BR, 1), lambda i, j: (i, 0)),
            pl.BlockSpec((_BR, 1), lambda i, j: (i + _HB, 0)),
            pl.BlockSpec((1, _NP), lambda i, j: (0, 0)),
            pl.BlockSpec((_NP, 2 * _F), lambda i, j: (0, 0)),
        ],
        out_specs=[
            pl.BlockSpec((_BR, _F), lambda i, j: (i, 0)),
            pl.BlockSpec((_BR, _F), lambda i, j: (i, 0)),
        ],
        out_shape=[
            jax.ShapeDtypeStruct((nhalf, _F), jnp.float32),
            jax.ShapeDtypeStruct((_N - nhalf, _F), jnp.float32),
        ],
        scratch_shapes=[
            pltpu.VMEM((_BR, 2 * _F), jnp.float32),
            pltpu.VMEM((_BR, 2 * _F), jnp.float32),
        ],
    )(adj, adj, s, s, t, out)

    return jnp.concatenate([yA, yB], axis=0)


# speedup vs baseline: 1.4533x; 1.4533x over previous
import functools

import jax
import jax.numpy as jnp
import numpy as np
from jax.experimental import pallas as pl
from jax.experimental.pallas import tpu as pltpu

_N = 10000
_F = 128
_ALPHA = 0.2

_NP = 10240
_BR = 512
_BC = 2560
_PR = 512
_HB = _NP // _BR // 2


def _project_kernel(x_ref, w_ref, b_ref, aw_ref, out_ref, st_ref):
    i = pl.program_id(0)
    o = jnp.dot(x_ref[...], w_ref[...], preferred_element_type=jnp.float32)
    o = o + b_ref[...]
    row = i * _PR + jax.lax.broadcasted_iota(jnp.int32, (_PR, 1), 0)
    o = jnp.where(row < _N, o, 0.0)
    col = jax.lax.broadcasted_iota(jnp.int32, (_PR, 2 * _F), 1)
    out_ref[...] = jnp.where(col < _F,
                             jnp.pad(o, ((0, 0), (0, _F))),
                             jnp.where(col == _F, 1.0, 0.0)).astype(jnp.bfloat16)
    st_ref[...] = jnp.dot(o, aw_ref[...], preferred_element_type=jnp.float32)


def _ev(adj, s, t):
    e = s + t
    e = jnp.maximum(e, jnp.bfloat16(_ALPHA) * e)
    return jnp.exp2(e) * adj.astype(jnp.bfloat16)


def _gat_kernel(adjA_ref, adjB_ref, sA_ref, sB_ref, t_ref, out_ref,
                yA_ref, yB_ref, accA_ref, accB_ref, *, nj):
    j = pl.program_id(1)

    @pl.when(j == 0)
    def _init():
        accA_ref[...] = jnp.zeros_like(accA_ref)
        accB_ref[...] = jnp.zeros_like(accB_ref)

    t = t_ref[:, pl.ds(j * _BC, _BC)]
    rhs = out_ref[pl.ds(j * _BC, _BC), :]
    evA = _ev(adjA_ref[...], sA_ref[...], t)
    evB = _ev(adjB_ref[...], sB_ref[...], t)

    @pl.when(j < nj - 1)
    def _acc_body():
        accA_ref[...] += jnp.dot(evA, rhs, preferred_element_type=jnp.float32)
        accB_ref[...] += jnp.dot(evB, rhs, preferred_element_type=jnp.float32)

    @pl.when(j == nj - 1)
    def _acc_last():
        col = j * _BC + jax.lax.broadcasted_iota(jnp.int32, (_BR, _BC), 1)
        mask = col < _N
        accA = accA_ref[...] + jnp.dot(jnp.where(mask, evA, jnp.bfloat16(0.0)), rhs,
                                       preferred_element_type=jnp.float32)
        accB = accB_ref[...] + jnp.dot(jnp.where(mask, evB, jnp.bfloat16(0.0)), rhs,
                                       preferred_element_type=jnp.float32)
        rsA = accA[:, _F:_F + 1]
        rsB = accB[:, _F:_F + 1]
        yA_ref[...] = accA[:, :_F] / jnp.where(rsA == 0.0, 1.0, rsA)
        yB_ref[...] = accB[:, :_F] / jnp.where(rsB == 0.0, 1.0, rsB)


def kernel(input, adj, W, b, attn_w):
    aw = attn_w.reshape(_F, 2) * np.float32(np.log2(np.e))
    b2 = b.reshape(1, _F)

    out, st = pl.pallas_call(
        _project_kernel,
        grid=(_NP // _PR,),
        in_specs=[
            pl.BlockSpec((_PR, _F), lambda i: (i, 0)),
            pl.BlockSpec((_F, _F), lambda i: (0, 0)),
            pl.BlockSpec((1, _F), lambda i: (0, 0)),
            pl.BlockSpec((_F, 2), lambda i: (0, 0)),
        ],
        out_specs=[
            pl.BlockSpec((_PR, 2 * _F), lambda i: (i, 0)),
            pl.BlockSpec((_PR, 2), lambda i: (i, 0)),
        ],
        out_shape=[
            jax.ShapeDtypeStruct((_NP, 2 * _F), jnp.bfloat16),
            jax.ShapeDtypeStruct((_NP, 2), jnp.float32),
        ],
    )(input, W, b2, aw)

    s = st[:, 0:1].astype(jnp.bfloat16)
    t = st[:, 1:2].T.astype(jnp.bfloat16)

    nhalf = _NP // 2
    ni, nj = _HB, _NP // _BC
    yA, yB = pl.pallas_call(
        functools.partial(_gat_kernel, nj=nj),
        grid=(ni, nj),
        in_specs=[
            pl.BlockSpec((_BR, _BC), lambda i, j: (i, j)),
            pl.BlockSpec((_BR, _BC), lambda i, j: (i + _HB, j)),
            pl.BlockSpec((_BR, 1), lambda i, j: (i, 0)),
            pl.BlockSpec((_BR, 1), lambda i, j: (i + _HB, 0)),
            pl.BlockSpec((1, _NP), lambda i, j: (0, 0)),
            pl.BlockSpec((_NP, 2 * _F), lambda i, j: (0, 0)),
        ],
        out_specs=[
            pl.BlockSpec((_BR, _F), lambda i, j: (i, 0)),
            pl.BlockSpec((_BR, _F), lambda i, j: (i, 0)),
        ],
        out_shape=[
            jax.ShapeDtypeStruct((nhalf, _F), jnp.float32),
            jax.ShapeDtypeStruct((_N - nhalf, _F), jnp.float32),
        ],
        scratch_shapes=[
            pltpu.VMEM((_BR, 2 * _F), jnp.float32),
            pltpu.VMEM((_BR, 2 * _F), jnp.float32),
        ],
    )(adj, adj, s, s, t, out)

    return jnp.concatenate([yA, yB], axis=0)
